# baseline (speedup 1.0000x reference)
import functools

import jax
import jax.numpy as jnp
from jax import lax
from jax.experimental import pallas as pl
from jax.experimental.pallas import tpu as pltpu
from jax.experimental.pallas import tpu_sc as plsc

N = 10000
E = 320000
D = 128
DO = 64
MU = 0.1
ALPHA = 1.0 / (1.0 + MU)
BETA = MU / (1.0 + MU)

NC = 2
NS = 16
NW = NC * NS
N_P = 10240
EB = 80
G = 4
E_TILE = 10240
E_PAD = NW * E_TILE
NB = E_TILE // EB
NGRP = NB // G
SLN = N_P // NS
ROWS_T = N_P // NS
RZ = 80

_mesh = plsc.VectorSubcoreMesh(core_axis_name="c", subcore_axis_name="s")

_GDN = lax.GatherDimensionNumbers(
    offset_dims=(), collapsed_slice_dims=(0,), start_index_map=(0,))


def _bcast_lane(vec, l):
    idx = jnp.full((16, 1), l, jnp.int32)
    return lax.gather(vec, idx, _GDN, slice_sizes=(1,),
                      mode=lax.GatherScatterMode.PROMISE_IN_BOUNDS)


def _zero_vec_ref(ref, n):
    for j in range(n // 16):
        ref[pl.ds(j * 16, 16)] = jnp.zeros((16,), jnp.float32)


@functools.partial(
    pl.kernel,
    out_type=jax.ShapeDtypeStruct((NC, N_P), jnp.float32),
    mesh=_mesh,
    compiler_params=pltpu.CompilerParams(needs_layout_passes=False),
    scratch_types=[
        pltpu.VMEM((NB, EB), jnp.int32),
        pltpu.VMEM((EB,), jnp.float32),
        pltpu.VMEM((SLN,), jnp.float32),
        pltpu.VMEM_SHARED((N_P,), jnp.float32),
        pltpu.SemaphoreType.DMA,
    ],
)
def _sc_degree(dstr_hbm, out_hbm, idx_d, ones_v, zbuf_v, acc_sh, sem):
    c = lax.axis_index("c")
    s = lax.axis_index("s")
    w = c * NS + s
    _zero_vec_ref(zbuf_v, SLN)
    pltpu.sync_copy(zbuf_v, acc_sh.at[pl.ds(s * SLN, SLN)])
    for j in range(EB // 16):
        ones_v[pl.ds(j * 16, 16)] = jnp.ones((16,), jnp.float32)
    pltpu.sync_copy(dstr_hbm.at[w], idx_d)
    plsc.subcore_barrier()
    descs = [pltpu.async_copy(ones_v, acc_sh.at[idx_d.at[i]], sem, add=True)
             for i in range(NB)]
    for dsc in descs:
        dsc.wait()
    plsc.subcore_barrier()
    pltpu.sync_copy(acc_sh.at[pl.ds(s * SLN, SLN)],
                    out_hbm.at[c, pl.ds(s * SLN, SLN)])


@functools.partial(
    pl.kernel,
    out_type=jax.ShapeDtypeStruct((NC, N_P), jnp.float32),
    mesh=_mesh,
    compiler_params=pltpu.CompilerParams(needs_layout_passes=False),
    scratch_types=[
        pltpu.VMEM((NB, EB), jnp.int32),
        pltpu.VMEM((NB, EB), jnp.int32),
        pltpu.VMEM((E_TILE,), jnp.float32),
        pltpu.VMEM((N_P,), jnp.float32),
        pltpu.VMEM((SLN,), jnp.float32),
        pltpu.VMEM_SHARED((N_P,), jnp.float32),
        pltpu.SemaphoreType.DMA,
    ],
)
def _sc_dsum(srcr_hbm, dstr_hbm, dinv_hbm, out_hbm,
             idx_s, idx_d, ew_v, dinv_v, zbuf_v, acc_sh, sem):
    c = lax.axis_index("c")
    s = lax.axis_index("s")
    w = c * NS + s
    _zero_vec_ref(zbuf_v, SLN)
    pltpu.sync_copy(zbuf_v, acc_sh.at[pl.ds(s * SLN, SLN)])
    pltpu.sync_copy(dinv_hbm, dinv_v)
    pltpu.sync_copy(srcr_hbm.at[w], idx_s)
    pltpu.sync_copy(dstr_hbm.at[w], idx_d)

    def body(i, carry):
        for g in range(EB // 16):
            sl = pl.ds(g * 16, 16)
            vs = plsc.load_gather(dinv_v, [idx_s[i, sl]])
            vd = plsc.load_gather(dinv_v, [idx_d[i, sl]])
            ew_v[pl.ds(i * EB + g * 16, 16)] = vs * vd
        return carry

    lax.fori_loop(0, NB, body, 0)
    plsc.subcore_barrier()
    descs = [pltpu.async_copy(ew_v.at[pl.ds(i * EB, EB)],
                              acc_sh.at[idx_s.at[i]], sem, add=True)
             for i in range(NB)]
    for dsc in descs:
        dsc.wait()
    plsc.subcore_barrier()
    pltpu.sync_copy(acc_sh.at[pl.ds(s * SLN, SLN)],
                    out_hbm.at[c, pl.ds(s * SLN, SLN)])


@functools.partial(
    pl.kernel,
    out_type=jax.ShapeDtypeStruct((NW, E_TILE), jnp.float32),
    mesh=_mesh,
    compiler_params=pltpu.CompilerParams(needs_layout_passes=False),
    scratch_types=[
        pltpu.VMEM((E_TILE,), jnp.int32),
        pltpu.VMEM((E_TILE,), jnp.int32),
        pltpu.VMEM((E_TILE,), jnp.float32),
        pltpu.VMEM((N_P,), jnp.float32),
    ],
)
def _sc_coef(srcf_hbm, dstf_hbm, q_hbm, out_hbm, idx_s, idx_d, cf_v, q_v):
    c = lax.axis_index("c")
    s = lax.axis_index("s")
    w = c * NS + s
    pltpu.sync_copy(q_hbm, q_v)
    pltpu.sync_copy(srcf_hbm.at[w], idx_s)
    pltpu.sync_copy(dstf_hbm.at[w], idx_d)

    def body(t, carry):
        sl = pl.ds(t * 16, 16)
        vs = plsc.load_gather(q_v, [idx_s[sl]])
        vd = plsc.load_gather(q_v, [idx_d[sl]])
        cf_v[sl] = vs * vd
        return carry

    lax.fori_loop(0, E_TILE // 16, body, 0)
    pltpu.sync_copy(cf_v, out_hbm.at[w])


@functools.partial(
    pl.kernel,
    out_type=jax.ShapeDtypeStruct((NC, N_P, D), jnp.float32),
    mesh=_mesh,
    compiler_params=pltpu.CompilerParams(needs_layout_passes=False),
    scratch_types=[
        pltpu.VMEM((G, EB), jnp.int32),
        pltpu.VMEM((G, EB), jnp.int32),
        pltpu.VMEM((G, EB), jnp.float32),
        pltpu.VMEM((G, EB), jnp.int32),
        pltpu.VMEM((G, EB), jnp.int32),
        pltpu.VMEM((G, EB), jnp.float32),
        pltpu.VMEM((G, EB, D), jnp.float32),
        pltpu.VMEM_SHARED((N_P, D), jnp.float32),
        pltpu.SemaphoreType.DMA,
        [pltpu.SemaphoreType.DMA] * G,
        [pltpu.SemaphoreType.DMA] * G,
    ],
)
def _sc_spmv(srcg_hbm, dstg_hbm, cfg_hbm, f_hbm, out_hbm,
             isA, idA, cfA, isB, idB, cfB, rows_v, acc_sh,
             semslab, semg, sems):
    c = lax.axis_index("c")
    s = lax.axis_index("s")
    w = c * NS + s
    nb = s * ROWS_T

    def zrow(r, carry):
        for j in range(D // 16):
            rows_v[0, r, pl.ds(j * 16, 16)] = jnp.zeros((16,), jnp.float32)
        return carry

    lax.fori_loop(0, RZ, zrow, 0)
    for k in range(ROWS_T // RZ):
        pltpu.sync_copy(rows_v.at[0], acc_sh.at[pl.ds(nb + k * RZ, RZ)])
    plsc.subcore_barrier()

    def prefetch(grp, is_t, id_t, cf_t):
        pltpu.async_copy(srcg_hbm.at[w, grp], is_t, semslab)
        pltpu.async_copy(dstg_hbm.at[w, grp], id_t, semslab)
        pltpu.async_copy(cfg_hbm.at[w, grp], cf_t, semslab)

    def wait_slabs(grp, is_t, id_t, cf_t):
        pltpu.make_async_copy(srcg_hbm.at[w, grp], is_t, semslab).wait()
        pltpu.make_async_copy(dstg_hbm.at[w, grp], id_t, semslab).wait()
        pltpu.make_async_copy(cfg_hbm.at[w, grp], cf_t, semslab).wait()

    def phase(grp, nxt, is_t, id_t, cf_t, pf_is, pf_id, pf_cf):
        wait_slabs(grp, is_t, id_t, cf_t)
        gd = [pltpu.async_copy(f_hbm.at[id_t.at[b]], rows_v.at[b], semg[b])
              for b in range(G)]
        prefetch(nxt, pf_is, pf_id, pf_cf)
        sd = []
        for b in range(G):
            gd[b].wait()

            def srow(g, carry, _b=b):
                cv = cf_t[_b, pl.ds(g * 16, 16)]
                for l in range(16):
                    cb = _bcast_lane(cv, l)
                    r = g * 16 + l
                    for j in range(D // 16):
                        sl = pl.ds(j * 16, 16)
                        rows_v[_b, r, sl] = rows_v[_b, r, sl] * cb
                return carry

            lax.fori_loop(0, EB // 16, srow, 0)
            sd.append(pltpu.async_copy(rows_v.at[b], acc_sh.at[is_t.at[b]],
                                       sems[b], add=True))
        for dsc in sd:
            dsc.wait()

    prefetch(0, isA, idA, cfA)

    def pair(k, carry):
        gA = 2 * k
        gB = 2 * k + 1
        gA2 = jnp.minimum(gA + 2, NGRP - 1)
        phase(gA, gB, isA, idA, cfA, isB, idB, cfB)
        phase(gB, gA2, isB, idB, cfB, isA, idA, cfA)
        return carry

    lax.fori_loop(0, NGRP // 2, pair, 0)
    wait_slabs(NGRP - 1, isA, idA, cfA)
    plsc.subcore_barrier()
    for k in range(ROWS_T // RZ):
        pltpu.sync_copy(acc_sh.at[pl.ds(nb + k * RZ, RZ)],
                        out_hbm.at[c, pl.ds(nb + k * RZ, RZ)])


def _tc_dinv_body(deg_ref, out_ref):
    out_ref[...] = lax.rsqrt(deg_ref[0] + deg_ref[1] + 1.0)


def _tc_dinv(deg2):
    return pl.pallas_call(
        _tc_dinv_body,
        out_shape=jax.ShapeDtypeStruct((N_P // D, D), jnp.float32),
    )(deg2.reshape(2, N_P // D, D))


def _tc_q_body(d2_ref, dinv_ref, q_ref, q2_ref):
    dinv = dinv_ref[...]
    dd = jnp.maximum(d2_ref[0] + d2_ref[1] + dinv * dinv, 1e-12)
    q = dinv * lax.rsqrt(dd)
    q_ref[...] = q
    q2_ref[...] = q * q


def _tc_q(d2, dinv):
    return pl.pallas_call(
        _tc_q_body,
        out_shape=(
            jax.ShapeDtypeStruct((N_P // D, D), jnp.float32),
            jax.ShapeDtypeStruct((N_P // D, D), jnp.float32),
        ),
    )(d2.reshape(2, N_P // D, D), dinv)


_RB_TC = 1280


def _tc_linrelu_body(x_ref, w_ref, b_ref, out_ref):
    y = jnp.dot(x_ref[...], w_ref[...], preferred_element_type=jnp.float32)
    out_ref[...] = jnp.maximum(y + b_ref[...], 0.0)


def _tc_linrelu(x, w, b):
    return pl.pallas_call(
        _tc_linrelu_body,
        out_shape=jax.ShapeDtypeStruct((N_P, D), jnp.float32),
        grid=(N_P // _RB_TC,),
        in_specs=[
            pl.BlockSpec((_RB_TC, D), lambda i: (i, 0)),
            pl.BlockSpec((D, D), lambda i: (0, 0)),
            pl.BlockSpec((1, D), lambda i: (0, 0)),
        ],
        out_specs=pl.BlockSpec((_RB_TC, D), lambda i: (i, 0)),
    )(x, w, b.reshape(1, D))


def _combine(g0, g1, f, f0, q2):
    return ALPHA * (g0 + g1 + q2 * f) + BETA * f0


def _tc_combine_body(g0_ref, g1_ref, f_ref, f0_ref, q2_ref, out_ref):
    out_ref[...] = _combine(g0_ref[...], g1_ref[...], f_ref[...],
                            f0_ref[...], q2_ref[...])


def _tc_combine(g0, g1, f, f0, q2):
    return pl.pallas_call(
        _tc_combine_body,
        out_shape=jax.ShapeDtypeStruct((N_P, D), jnp.float32),
        grid=(N_P // _RB_TC,),
        in_specs=[
            pl.BlockSpec((_RB_TC, D), lambda i: (i, 0)),
            pl.BlockSpec((_RB_TC, D), lambda i: (i, 0)),
            pl.BlockSpec((_RB_TC, D), lambda i: (i, 0)),
            pl.BlockSpec((_RB_TC, D), lambda i: (i, 0)),
            pl.BlockSpec((_RB_TC, 1), lambda i: (i, 0)),
        ],
        out_specs=pl.BlockSpec((_RB_TC, D), lambda i: (i, 0)),
    )(g0, g1, f, f0, q2)


def _tc_comb_mm_relu_body(g0_ref, g1_ref, f_ref, f0_ref, q2_ref, w_ref, b_ref,
                          out_ref):
    z = _combine(g0_ref[...], g1_ref[...], f_ref[...], f0_ref[...], q2_ref[...])
    y = jnp.dot(z, w_ref[...], preferred_element_type=jnp.float32)
    out_ref[...] = jnp.maximum(y + b_ref[...], 0.0)


def _tc_comb_mm_relu(g0, g1, f, f0, q2, w, b):
    return pl.pallas_call(
        _tc_comb_mm_relu_body,
        out_shape=jax.ShapeDtypeStruct((N_P, D), jnp.float32),
        grid=(N_P // _RB_TC,),
        in_specs=[
            pl.BlockSpec((_RB_TC, D), lambda i: (i, 0)),
            pl.BlockSpec((_RB_TC, D), lambda i: (i, 0)),
            pl.BlockSpec((_RB_TC, D), lambda i: (i, 0)),
            pl.BlockSpec((_RB_TC, D), lambda i: (i, 0)),
            pl.BlockSpec((_RB_TC, 1), lambda i: (i, 0)),
            pl.BlockSpec((D, D), lambda i: (0, 0)),
            pl.BlockSpec((1, D), lambda i: (0, 0)),
        ],
        out_specs=pl.BlockSpec((_RB_TC, D), lambda i: (i, 0)),
    )(g0, g1, f, f0, q2, w, b.reshape(1, D))


_RB_LSM = 2000


def _tc_comb_mm_lsm_body(g0_ref, g1_ref, f_ref, f0_ref, q2_ref, w_ref, b_ref,
                         out_ref):
    z = _combine(g0_ref[...], g1_ref[...], f_ref[...], f0_ref[...], q2_ref[...])
    y = jnp.dot(z, w_ref[...], preferred_element_type=jnp.float32) + b_ref[...]
    m = jnp.max(y, axis=1, keepdims=True)
    lse = m + jnp.log(jnp.sum(jnp.exp(y - m), axis=1, keepdims=True))
    out_ref[...] = y - lse


def _tc_comb_mm_lsm(g0, g1, f, f0, q2, w, b):
    return pl.pallas_call(
        _tc_comb_mm_lsm_body,
        out_shape=jax.ShapeDtypeStruct((N, DO), jnp.float32),
        grid=(N // _RB_LSM,),
        in_specs=[
            pl.BlockSpec((_RB_LSM, D), lambda i: (i, 0)),
            pl.BlockSpec((_RB_LSM, D), lambda i: (i, 0)),
            pl.BlockSpec((_RB_LSM, D), lambda i: (i, 0)),
            pl.BlockSpec((_RB_LSM, D), lambda i: (i, 0)),
            pl.BlockSpec((_RB_LSM, 1), lambda i: (i, 0)),
            pl.BlockSpec((D, DO), lambda i: (0, 0)),
            pl.BlockSpec((1, DO), lambda i: (0, 0)),
        ],
        out_specs=pl.BlockSpec((_RB_LSM, DO), lambda i: (i, 0)),
    )(g0, g1, f, f0, q2, w, b.reshape(1, DO))


def kernel(x, edge_index, W_l1, b_l1, W_c1, b_c1, W_c2, b_c2):
    pad = jnp.full((E_PAD - E,), N, jnp.int32)
    src = jnp.concatenate([edge_index[0].astype(jnp.int32), pad])
    dst = jnp.concatenate([edge_index[1].astype(jnp.int32), pad])
    srcr = src.reshape(NW, NB, EB)
    dstr = dst.reshape(NW, NB, EB)
    srcg = src.reshape(NW, NGRP, G, EB)
    dstg = dst.reshape(NW, NGRP, G, EB)
    srcf = src.reshape(NW, E_TILE)
    dstf = dst.reshape(NW, E_TILE)
    xp = jnp.concatenate([x, jnp.zeros((N_P - N, D), jnp.float32)])

    deg2 = _sc_degree(dstr)
    dinv = _tc_dinv(deg2)
    d2 = _sc_dsum(srcr, dstr, dinv.reshape(N_P))
    q, q2f = _tc_q(d2, dinv)
    coef = _sc_coef(srcf, dstf, q.reshape(N_P))
    cfg = coef.reshape(NW, NGRP, G, EB)
    q2 = q2f.reshape(N_P, 1)

    f0 = _tc_linrelu(xp, W_l1, b_l1)

    g = _sc_spmv(srcg, dstg, cfg, f0)
    f1 = _tc_combine(g[0], g[1], f0, f0, q2)
    g = _sc_spmv(srcg, dstg, cfg, f1)
    h = _tc_comb_mm_relu(g[0], g[1], f1, f0, q2, W_c1, b_c1)

    g = _sc_spmv(srcg, dstg, cfg, h)
    f1 = _tc_combine(g[0], g[1], h, h, q2)
    g = _sc_spmv(srcg, dstg, cfg, f1)
    out = _tc_comb_mm_lsm(g[0], g[1], f1, h, q2, W_c2, b_c2)
    return out


# speedup vs baseline: 2.5999x; 2.5999x over previous
import functools

import jax
import jax.numpy as jnp
from jax import lax
from jax.experimental import pallas as pl
from jax.experimental.pallas import tpu as pltpu
from jax.experimental.pallas import tpu_sc as plsc

N = 10000
E = 320000
D = 128
DO = 64
MU = 0.1
ALPHA = 1.0 / (1.0 + MU)
BETA = MU / (1.0 + MU)

NC = 2
NS = 16
NW = NC * NS
N_P = 10240
EB = 80
G = 4
E_TILE = 10240
E_PAD = NW * E_TILE
NB = E_TILE // EB
NGRP = NB // G
SLN = N_P // NS
ROWS_T = N_P // NS
RZ = 80

_mesh = plsc.VectorSubcoreMesh(core_axis_name="c", subcore_axis_name="s")

_GDN = lax.GatherDimensionNumbers(
    offset_dims=(), collapsed_slice_dims=(0,), start_index_map=(0,))


def _bcast_lane(vec, l):
    idx = jnp.full((16, 1), l, jnp.int32)
    return lax.gather(vec, idx, _GDN, slice_sizes=(1,),
                      mode=lax.GatherScatterMode.PROMISE_IN_BOUNDS)


def _zero_vec_ref(ref, n):
    for j in range(n // 16):
        ref[pl.ds(j * 16, 16)] = jnp.zeros((16,), jnp.float32)


@functools.partial(
    pl.kernel,
    out_type=jax.ShapeDtypeStruct((NC, N_P), jnp.float32),
    mesh=_mesh,
    compiler_params=pltpu.CompilerParams(needs_layout_passes=False),
    scratch_types=[
        pltpu.VMEM((NB, EB), jnp.int32),
        pltpu.VMEM((EB,), jnp.float32),
        pltpu.VMEM((SLN,), jnp.float32),
        pltpu.VMEM_SHARED((N_P,), jnp.float32),
        pltpu.SemaphoreType.DMA,
    ],
)
def _sc_degree(dstr_hbm, out_hbm, idx_d, ones_v, zbuf_v, acc_sh, sem):
    c = lax.axis_index("c")
    s = lax.axis_index("s")
    w = c * NS + s
    _zero_vec_ref(zbuf_v, SLN)
    pltpu.sync_copy(zbuf_v, acc_sh.at[pl.ds(s * SLN, SLN)])
    for j in range(EB // 16):
        ones_v[pl.ds(j * 16, 16)] = jnp.ones((16,), jnp.float32)
    pltpu.sync_copy(dstr_hbm.at[w], idx_d)
    plsc.subcore_barrier()
    descs = [pltpu.async_copy(ones_v, acc_sh.at[idx_d.at[i]], sem, add=True)
             for i in range(NB)]
    for dsc in descs:
        dsc.wait()
    plsc.subcore_barrier()
    pltpu.sync_copy(acc_sh.at[pl.ds(s * SLN, SLN)],
                    out_hbm.at[c, pl.ds(s * SLN, SLN)])


@functools.partial(
    pl.kernel,
    out_type=jax.ShapeDtypeStruct((NC, N_P), jnp.float32),
    mesh=_mesh,
    compiler_params=pltpu.CompilerParams(needs_layout_passes=False),
    scratch_types=[
        pltpu.VMEM((NB, EB), jnp.int32),
        pltpu.VMEM((NB, EB), jnp.int32),
        pltpu.VMEM((E_TILE,), jnp.float32),
        pltpu.VMEM((N_P,), jnp.float32),
        pltpu.VMEM((SLN,), jnp.float32),
        pltpu.VMEM_SHARED((N_P,), jnp.float32),
        pltpu.SemaphoreType.DMA,
    ],
)
def _sc_dsum(srcr_hbm, dstr_hbm, dinv_hbm, out_hbm,
             idx_s, idx_d, ew_v, dinv_v, zbuf_v, acc_sh, sem):
    c = lax.axis_index("c")
    s = lax.axis_index("s")
    w = c * NS + s
    _zero_vec_ref(zbuf_v, SLN)
    pltpu.sync_copy(zbuf_v, acc_sh.at[pl.ds(s * SLN, SLN)])
    pltpu.sync_copy(dinv_hbm, dinv_v)
    pltpu.sync_copy(srcr_hbm.at[w], idx_s)
    pltpu.sync_copy(dstr_hbm.at[w], idx_d)

    def body(i, carry):
        for g in range(EB // 16):
            sl = pl.ds(g * 16, 16)
            vs = plsc.load_gather(dinv_v, [idx_s[i, sl]])
            vd = plsc.load_gather(dinv_v, [idx_d[i, sl]])
            ew_v[pl.ds(i * EB + g * 16, 16)] = vs * vd
        return carry

    lax.fori_loop(0, NB, body, 0)
    plsc.subcore_barrier()
    descs = [pltpu.async_copy(ew_v.at[pl.ds(i * EB, EB)],
                              acc_sh.at[idx_s.at[i]], sem, add=True)
             for i in range(NB)]
    for dsc in descs:
        dsc.wait()
    plsc.subcore_barrier()
    pltpu.sync_copy(acc_sh.at[pl.ds(s * SLN, SLN)],
                    out_hbm.at[c, pl.ds(s * SLN, SLN)])


@functools.partial(
    pl.kernel,
    out_type=jax.ShapeDtypeStruct((NW, E_TILE), jnp.float32),
    mesh=_mesh,
    compiler_params=pltpu.CompilerParams(needs_layout_passes=False),
    scratch_types=[
        pltpu.VMEM((E_TILE,), jnp.int32),
        pltpu.VMEM((E_TILE,), jnp.int32),
        pltpu.VMEM((E_TILE,), jnp.float32),
        pltpu.VMEM((N_P,), jnp.float32),
    ],
)
def _sc_coef(srcf_hbm, dstf_hbm, q_hbm, out_hbm, idx_s, idx_d, cf_v, q_v):
    c = lax.axis_index("c")
    s = lax.axis_index("s")
    w = c * NS + s
    pltpu.sync_copy(q_hbm, q_v)
    pltpu.sync_copy(srcf_hbm.at[w], idx_s)
    pltpu.sync_copy(dstf_hbm.at[w], idx_d)

    def body(t, carry):
        sl = pl.ds(t * 16, 16)
        vs = plsc.load_gather(q_v, [idx_s[sl]])
        vd = plsc.load_gather(q_v, [idx_d[sl]])
        cf_v[sl] = vs * vd
        return carry

    lax.fori_loop(0, E_TILE // 16, body, 0)
    pltpu.sync_copy(cf_v, out_hbm.at[w])


@functools.partial(
    pl.kernel,
    out_type=jax.ShapeDtypeStruct((NC, N_P, D), jnp.float32),
    mesh=_mesh,
    compiler_params=pltpu.CompilerParams(needs_layout_passes=False),
    scratch_types=[
        pltpu.VMEM((G, EB), jnp.int32),
        pltpu.VMEM((G, EB), jnp.int32),
        pltpu.VMEM((G, EB), jnp.float32),
        pltpu.VMEM((G, EB), jnp.int32),
        pltpu.VMEM((G, EB), jnp.int32),
        pltpu.VMEM((G, EB), jnp.float32),
        pltpu.VMEM((G, EB, D), jnp.float32),
        pltpu.VMEM_SHARED((N_P, D), jnp.float32),
        pltpu.SemaphoreType.DMA,
        [pltpu.SemaphoreType.DMA] * G,
        [pltpu.SemaphoreType.DMA] * G,
    ],
)
def _sc_spmv(srcg_hbm, dstg_hbm, cfg_hbm, f_hbm, out_hbm,
             isA, idA, cfA, isB, idB, cfB, rows_v, acc_sh,
             semslab, semg, sems):
    c = lax.axis_index("c")
    s = lax.axis_index("s")
    w = c * NS + s
    nb = s * ROWS_T

    def zrow(r, carry):
        for j in range(D // 16):
            rows_v[0, r, pl.ds(j * 16, 16)] = jnp.zeros((16,), jnp.float32)
        return carry

    lax.fori_loop(0, RZ, zrow, 0)
    for k in range(ROWS_T // RZ):
        pltpu.sync_copy(rows_v.at[0], acc_sh.at[pl.ds(nb + k * RZ, RZ)])
    plsc.subcore_barrier()

    def prefetch(grp, is_t, id_t, cf_t):
        pltpu.async_copy(srcg_hbm.at[w, grp], is_t, semslab)
        pltpu.async_copy(dstg_hbm.at[w, grp], id_t, semslab)
        pltpu.async_copy(cfg_hbm.at[w, grp], cf_t, semslab)

    def wait_slabs(grp, is_t, id_t, cf_t):
        pltpu.make_async_copy(srcg_hbm.at[w, grp], is_t, semslab).wait()
        pltpu.make_async_copy(dstg_hbm.at[w, grp], id_t, semslab).wait()
        pltpu.make_async_copy(cfg_hbm.at[w, grp], cf_t, semslab).wait()

    def phase(grp, nxt, is_t, id_t, cf_t, pf_is, pf_id, pf_cf):
        wait_slabs(grp, is_t, id_t, cf_t)
        gd = [pltpu.async_copy(f_hbm.at[id_t.at[b]], rows_v.at[b], semg[b])
              for b in range(G)]
        prefetch(nxt, pf_is, pf_id, pf_cf)
        sd = []
        for b in range(G):
            gd[b].wait()

            def srow(g, carry, _b=b):
                cv = cf_t[_b, pl.ds(g * 16, 16)]
                for l in range(16):
                    cb = _bcast_lane(cv, l)
                    r = g * 16 + l
                    for j in range(D // 16):
                        sl = pl.ds(j * 16, 16)
                        rows_v[_b, r, sl] = rows_v[_b, r, sl] * cb
                return carry

            lax.fori_loop(0, EB // 16, srow, 0)
            sd.append(pltpu.async_copy(rows_v.at[b], acc_sh.at[is_t.at[b]],
                                       sems[b], add=True))
        for dsc in sd:
            dsc.wait()

    prefetch(0, isA, idA, cfA)

    def pair(k, carry):
        gA = 2 * k
        gB = 2 * k + 1
        gA2 = jnp.minimum(gA + 2, NGRP - 1)
        phase(gA, gB, isA, idA, cfA, isB, idB, cfB)
        phase(gB, gA2, isB, idB, cfB, isA, idA, cfA)
        return carry

    lax.fori_loop(0, NGRP // 2, pair, 0)
    wait_slabs(NGRP - 1, isA, idA, cfA)
    plsc.subcore_barrier()
    for k in range(ROWS_T // RZ):
        pltpu.sync_copy(acc_sh.at[pl.ds(nb + k * RZ, RZ)],
                        out_hbm.at[c, pl.ds(nb + k * RZ, RZ)])


def _tc_dinv_body(deg_ref, out_ref):
    out_ref[...] = lax.rsqrt(deg_ref[0] + deg_ref[1] + 1.0)


def _tc_dinv(deg2):
    return pl.pallas_call(
        _tc_dinv_body,
        out_shape=jax.ShapeDtypeStruct((N_P // D, D), jnp.float32),
    )(deg2.reshape(2, N_P // D, D))


def _tc_q_body(d2_ref, dinv_ref, q_ref, q2_ref):
    dinv = dinv_ref[...]
    dd = jnp.maximum(d2_ref[0] + d2_ref[1] + dinv * dinv, 1e-12)
    q = dinv * lax.rsqrt(dd)
    q_ref[...] = q
    q2_ref[...] = q * q


def _tc_q(d2, dinv):
    return pl.pallas_call(
        _tc_q_body,
        out_shape=(
            jax.ShapeDtypeStruct((N_P // D, D), jnp.float32),
            jax.ShapeDtypeStruct((N_P // D, D), jnp.float32),
        ),
    )(d2.reshape(2, N_P // D, D), dinv)


_RB_TC = 1280


def _tc_linrelu_body(x_ref, w_ref, b_ref, out_ref):
    y = jnp.dot(x_ref[...], w_ref[...], preferred_element_type=jnp.float32)
    out_ref[...] = jnp.maximum(y + b_ref[...], 0.0)


def _tc_linrelu(x, w, b):
    return pl.pallas_call(
        _tc_linrelu_body,
        out_shape=jax.ShapeDtypeStruct((N_P, D), jnp.float32),
        grid=(N_P // _RB_TC,),
        in_specs=[
            pl.BlockSpec((_RB_TC, D), lambda i: (i, 0)),
            pl.BlockSpec((D, D), lambda i: (0, 0)),
            pl.BlockSpec((1, D), lambda i: (0, 0)),
        ],
        out_specs=pl.BlockSpec((_RB_TC, D), lambda i: (i, 0)),
    )(x, w, b.reshape(1, D))


def _combine(g0, g1, f, f0, q2):
    return ALPHA * (g0 + g1 + q2 * f) + BETA * f0


def _tc_combine_body(g0_ref, g1_ref, f_ref, f0_ref, q2_ref, out_ref):
    out_ref[...] = _combine(g0_ref[...], g1_ref[...], f_ref[...],
                            f0_ref[...], q2_ref[...])


def _tc_combine(g0, g1, f, f0, q2):
    return pl.pallas_call(
        _tc_combine_body,
        out_shape=jax.ShapeDtypeStruct((N_P, D), jnp.float32),
        grid=(N_P // _RB_TC,),
        in_specs=[
            pl.BlockSpec((_RB_TC, D), lambda i: (i, 0)),
            pl.BlockSpec((_RB_TC, D), lambda i: (i, 0)),
            pl.BlockSpec((_RB_TC, D), lambda i: (i, 0)),
            pl.BlockSpec((_RB_TC, D), lambda i: (i, 0)),
            pl.BlockSpec((_RB_TC, 1), lambda i: (i, 0)),
        ],
        out_specs=pl.BlockSpec((_RB_TC, D), lambda i: (i, 0)),
    )(g0, g1, f, f0, q2)


def _tc_comb_mm_relu_body(g0_ref, g1_ref, f_ref, f0_ref, q2_ref, w_ref, b_ref,
                          out_ref):
    z = _combine(g0_ref[...], g1_ref[...], f_ref[...], f0_ref[...], q2_ref[...])
    y = jnp.dot(z, w_ref[...], preferred_element_type=jnp.float32)
    out_ref[...] = jnp.maximum(y + b_ref[...], 0.0)


def _tc_comb_mm_relu(g0, g1, f, f0, q2, w, b):
    return pl.pallas_call(
        _tc_comb_mm_relu_body,
        out_shape=jax.ShapeDtypeStruct((N_P, D), jnp.float32),
        grid=(N_P // _RB_TC,),
        in_specs=[
            pl.BlockSpec((_RB_TC, D), lambda i: (i, 0)),
            pl.BlockSpec((_RB_TC, D), lambda i: (i, 0)),
            pl.BlockSpec((_RB_TC, D), lambda i: (i, 0)),
            pl.BlockSpec((_RB_TC, D), lambda i: (i, 0)),
            pl.BlockSpec((_RB_TC, 1), lambda i: (i, 0)),
            pl.BlockSpec((D, D), lambda i: (0, 0)),
            pl.BlockSpec((1, D), lambda i: (0, 0)),
        ],
        out_specs=pl.BlockSpec((_RB_TC, D), lambda i: (i, 0)),
    )(g0, g1, f, f0, q2, w, b.reshape(1, D))


_RB_LSM = 2000


def _tc_comb_mm_lsm_body(g0_ref, g1_ref, f_ref, f0_ref, q2_ref, w_ref, b_ref,
                         out_ref):
    z = _combine(g0_ref[...], g1_ref[...], f_ref[...], f0_ref[...], q2_ref[...])
    y = jnp.dot(z, w_ref[...], preferred_element_type=jnp.float32) + b_ref[...]
    m = jnp.max(y, axis=1, keepdims=True)
    lse = m + jnp.log(jnp.sum(jnp.exp(y - m), axis=1, keepdims=True))
    out_ref[...] = y - lse


def _tc_comb_mm_lsm(g0, g1, f, f0, q2, w, b):
    return pl.pallas_call(
        _tc_comb_mm_lsm_body,
        out_shape=jax.ShapeDtypeStruct((N, DO), jnp.float32),
        grid=(N // _RB_LSM,),
        in_specs=[
            pl.BlockSpec((_RB_LSM, D), lambda i: (i, 0)),
            pl.BlockSpec((_RB_LSM, D), lambda i: (i, 0)),
            pl.BlockSpec((_RB_LSM, D), lambda i: (i, 0)),
            pl.BlockSpec((_RB_LSM, D), lambda i: (i, 0)),
            pl.BlockSpec((_RB_LSM, 1), lambda i: (i, 0)),
            pl.BlockSpec((D, DO), lambda i: (0, 0)),
            pl.BlockSpec((1, DO), lambda i: (0, 0)),
        ],
        out_specs=pl.BlockSpec((_RB_LSM, DO), lambda i: (i, 0)),
    )(g0, g1, f, f0, q2, w, b.reshape(1, DO))


def kernel(x, edge_index, W_l1, b_l1, W_c1, b_c1, W_c2, b_c2):
    pad = (jnp.arange(E_PAD - E, dtype=jnp.int32) % (N_P - N)) + N
    src = jnp.concatenate([edge_index[0].astype(jnp.int32), pad])
    dst = jnp.concatenate([edge_index[1].astype(jnp.int32), pad])
    srcr = src.reshape(NW, NB, EB)
    dstr = dst.reshape(NW, NB, EB)
    srcg = src.reshape(NW, NGRP, G, EB)
    dstg = dst.reshape(NW, NGRP, G, EB)
    srcf = src.reshape(NW, E_TILE)
    dstf = dst.reshape(NW, E_TILE)
    xp = jnp.concatenate([x, jnp.zeros((N_P - N, D), jnp.float32)])

    deg2 = _sc_degree(dstr)
    dinv = _tc_dinv(deg2)
    d2 = _sc_dsum(srcr, dstr, dinv.reshape(N_P))
    q, q2f = _tc_q(d2, dinv)
    coef = _sc_coef(srcf, dstf, q.reshape(N_P))
    cfg = coef.reshape(NW, NGRP, G, EB)
    q2 = q2f.reshape(N_P, 1)

    f0 = _tc_linrelu(xp, W_l1, b_l1)

    g = _sc_spmv(srcg, dstg, cfg, f0)
    f1 = _tc_combine(g[0], g[1], f0, f0, q2)
    g = _sc_spmv(srcg, dstg, cfg, f1)
    h = _tc_comb_mm_relu(g[0], g[1], f1, f0, q2, W_c1, b_c1)

    g = _sc_spmv(srcg, dstg, cfg, h)
    f1 = _tc_combine(g[0], g[1], h, h, q2)
    g = _sc_spmv(srcg, dstg, cfg, f1)
    out = _tc_comb_mm_lsm(g[0], g[1], f1, h, q2, W_c2, b_c2)
    return out


# speedup vs baseline: 3.0448x; 1.1711x over previous
import functools

import jax
import jax.numpy as jnp
from jax import lax
from jax.experimental import pallas as pl
from jax.experimental.pallas import tpu as pltpu
from jax.experimental.pallas import tpu_sc as plsc

N = 10000
E = 320000
D = 128
DO = 64
MU = 0.1
ALPHA = 1.0 / (1.0 + MU)
BETA = MU / (1.0 + MU)

NC = 2
NS = 16
NW = NC * NS
N_P = 10240
EB = 80
G = 4
E_TILE = 10240
E_PAD = NW * E_TILE
NB = E_TILE // EB
NGRP = NB // G
SLN = N_P // NS
ROWS_T = N_P // NS
RZ = 80

_mesh = plsc.VectorSubcoreMesh(core_axis_name="c", subcore_axis_name="s")

_GDN = lax.GatherDimensionNumbers(
    offset_dims=(), collapsed_slice_dims=(0,), start_index_map=(0,))


def _bcast_lane(vec, l):
    idx = jnp.full((16, 1), l, jnp.int32)
    return lax.gather(vec, idx, _GDN, slice_sizes=(1,),
                      mode=lax.GatherScatterMode.PROMISE_IN_BOUNDS)


def _zero_vec_ref(ref, n):
    for j in range(n // 16):
        ref[pl.ds(j * 16, 16)] = jnp.zeros((16,), jnp.float32)


@functools.partial(
    pl.kernel,
    out_type=jax.ShapeDtypeStruct((NC, N_P), jnp.float32),
    mesh=_mesh,
    compiler_params=pltpu.CompilerParams(needs_layout_passes=False),
    scratch_types=[
        pltpu.VMEM((NB, EB), jnp.int32),
        pltpu.VMEM((EB,), jnp.float32),
        pltpu.VMEM((SLN,), jnp.float32),
        pltpu.VMEM_SHARED((N_P,), jnp.float32),
        pltpu.SemaphoreType.DMA,
    ],
)
def _sc_degree(dstr_hbm, out_hbm, idx_d, ones_v, zbuf_v, acc_sh, sem):
    c = lax.axis_index("c")
    s = lax.axis_index("s")
    w = c * NS + s
    _zero_vec_ref(zbuf_v, SLN)
    pltpu.sync_copy(zbuf_v, acc_sh.at[pl.ds(s * SLN, SLN)])
    for j in range(EB // 16):
        ones_v[pl.ds(j * 16, 16)] = jnp.ones((16,), jnp.float32)
    pltpu.sync_copy(dstr_hbm.at[w], idx_d)
    plsc.subcore_barrier()
    descs = [pltpu.async_copy(ones_v, acc_sh.at[idx_d.at[i]], sem, add=True)
             for i in range(NB)]
    for dsc in descs:
        dsc.wait()
    plsc.subcore_barrier()
    pltpu.sync_copy(acc_sh.at[pl.ds(s * SLN, SLN)],
                    out_hbm.at[c, pl.ds(s * SLN, SLN)])


@functools.partial(
    pl.kernel,
    out_type=jax.ShapeDtypeStruct((NC, N_P), jnp.float32),
    mesh=_mesh,
    compiler_params=pltpu.CompilerParams(needs_layout_passes=False),
    scratch_types=[
        pltpu.VMEM((NB, EB), jnp.int32),
        pltpu.VMEM((NB, EB), jnp.int32),
        pltpu.VMEM((E_TILE,), jnp.float32),
        pltpu.VMEM((N_P,), jnp.float32),
        pltpu.VMEM((SLN,), jnp.float32),
        pltpu.VMEM_SHARED((N_P,), jnp.float32),
        pltpu.SemaphoreType.DMA,
    ],
)
def _sc_dsum(srcr_hbm, dstr_hbm, dinv_hbm, out_hbm,
             idx_s, idx_d, ew_v, dinv_v, zbuf_v, acc_sh, sem):
    c = lax.axis_index("c")
    s = lax.axis_index("s")
    w = c * NS + s
    _zero_vec_ref(zbuf_v, SLN)
    pltpu.sync_copy(zbuf_v, acc_sh.at[pl.ds(s * SLN, SLN)])
    pltpu.sync_copy(dinv_hbm, dinv_v)
    pltpu.sync_copy(srcr_hbm.at[w], idx_s)
    pltpu.sync_copy(dstr_hbm.at[w], idx_d)

    def body(i, carry):
        for g in range(EB // 16):
            sl = pl.ds(g * 16, 16)
            vs = plsc.load_gather(dinv_v, [idx_s[i, sl]])
            vd = plsc.load_gather(dinv_v, [idx_d[i, sl]])
            ew_v[pl.ds(i * EB + g * 16, 16)] = vs * vd
        return carry

    lax.fori_loop(0, NB, body, 0)
    plsc.subcore_barrier()
    descs = [pltpu.async_copy(ew_v.at[pl.ds(i * EB, EB)],
                              acc_sh.at[idx_s.at[i]], sem, add=True)
             for i in range(NB)]
    for dsc in descs:
        dsc.wait()
    plsc.subcore_barrier()
    pltpu.sync_copy(acc_sh.at[pl.ds(s * SLN, SLN)],
                    out_hbm.at[c, pl.ds(s * SLN, SLN)])


@functools.partial(
    pl.kernel,
    out_type=jax.ShapeDtypeStruct((NW, E_TILE), jnp.float32),
    mesh=_mesh,
    compiler_params=pltpu.CompilerParams(needs_layout_passes=False),
    scratch_types=[
        pltpu.VMEM((E_TILE,), jnp.int32),
        pltpu.VMEM((E_TILE,), jnp.int32),
        pltpu.VMEM((E_TILE,), jnp.float32),
        pltpu.VMEM((N_P,), jnp.float32),
    ],
)
def _sc_coef(srcf_hbm, dstf_hbm, q_hbm, out_hbm, idx_s, idx_d, cf_v, q_v):
    c = lax.axis_index("c")
    s = lax.axis_index("s")
    w = c * NS + s
    pltpu.sync_copy(q_hbm, q_v)
    pltpu.sync_copy(srcf_hbm.at[w], idx_s)
    pltpu.sync_copy(dstf_hbm.at[w], idx_d)

    def body(t, carry):
        sl = pl.ds(t * 16, 16)
        vs = plsc.load_gather(q_v, [idx_s[sl]])
        vd = plsc.load_gather(q_v, [idx_d[sl]])
        cf_v[sl] = vs * vd
        return carry

    lax.fori_loop(0, E_TILE // 16, body, 0)
    pltpu.sync_copy(cf_v, out_hbm.at[w])


@functools.partial(
    pl.kernel,
    out_type=jax.ShapeDtypeStruct((NC, N_P, D), jnp.float32),
    mesh=_mesh,
    compiler_params=pltpu.CompilerParams(needs_layout_passes=False),
    scratch_types=[
        pltpu.VMEM((G, EB), jnp.int32),
        pltpu.VMEM((G, EB), jnp.int32),
        pltpu.VMEM((G, EB), jnp.float32),
        pltpu.VMEM((G, EB), jnp.int32),
        pltpu.VMEM((G, EB), jnp.int32),
        pltpu.VMEM((G, EB), jnp.float32),
        pltpu.VMEM((G, EB, D), jnp.float32),
        pltpu.VMEM_SHARED((N_P, D), jnp.float32),
        pltpu.SemaphoreType.DMA,
        pltpu.SemaphoreType.DMA,
        [pltpu.SemaphoreType.DMA] * G,
        [pltpu.SemaphoreType.DMA] * G,
    ],
)
def _sc_spmv(srcg_hbm, dstg_hbm, cfg_hbm, f_hbm, out_hbm,
             isA, idA, cfA, isB, idB, cfB, rows_v, acc_sh,
             semslab, semz, semg, sems):
    c = lax.axis_index("c")
    s = lax.axis_index("s")
    w = c * NS + s
    nb = s * ROWS_T

    for b in range(G):
        def zrow(r, carry, _b=b):
            for j in range(D // 16):
                rows_v[_b, r, pl.ds(j * 16, 16)] = jnp.zeros((16,),
                                                             jnp.float32)
            return carry

        lax.fori_loop(0, EB, zrow, 0)
    for b in range(G):
        for t in range(EB // 16):
            isA[b, pl.ds(t * 16, 16)] = lax.iota(jnp.int32, 16) + 16 * t
    prime = [pltpu.async_copy(rows_v.at[b], acc_sh.at[isA.at[b]], sems[b],
                              add=True)
             for b in range(G)]
    del prime
    zd = [pltpu.async_copy(rows_v.at[0], acc_sh.at[pl.ds(nb + k * RZ, RZ)],
                           semz)
          for k in range(ROWS_T // RZ)]
    for dsc in zd:
        dsc.wait()
    plsc.subcore_barrier()

    def prefetch(grp, is_t, id_t, cf_t):
        pltpu.async_copy(srcg_hbm.at[w, grp], is_t, semslab)
        pltpu.async_copy(dstg_hbm.at[w, grp], id_t, semslab)
        pltpu.async_copy(cfg_hbm.at[w, grp], cf_t, semslab)

    def wait_slabs(grp, is_t, id_t, cf_t):
        pltpu.make_async_copy(srcg_hbm.at[w, grp], is_t, semslab).wait()
        pltpu.make_async_copy(dstg_hbm.at[w, grp], id_t, semslab).wait()
        pltpu.make_async_copy(cfg_hbm.at[w, grp], cf_t, semslab).wait()

    def phase(grp, nxt, is_t, id_t, cf_t, pf_is, pf_id, pf_cf):
        wait_slabs(grp, is_t, id_t, cf_t)
        gd = []
        for b in range(G):
            pltpu.make_async_copy(rows_v.at[b], acc_sh.at[is_t.at[b]],
                                  sems[b]).wait()
            gd.append(pltpu.async_copy(f_hbm.at[id_t.at[b]], rows_v.at[b],
                                       semg[b]))
        prefetch(nxt, pf_is, pf_id, pf_cf)
        for b in range(G):
            gd[b].wait()

            def srow(g, carry, _b=b):
                cv = cf_t[_b, pl.ds(g * 16, 16)]
                for l in range(16):
                    cb = _bcast_lane(cv, l)
                    r = g * 16 + l
                    for j in range(D // 16):
                        sl = pl.ds(j * 16, 16)
                        rows_v[_b, r, sl] = rows_v[_b, r, sl] * cb
                return carry

            lax.fori_loop(0, EB // 16, srow, 0)
            pltpu.async_copy(rows_v.at[b], acc_sh.at[is_t.at[b]], sems[b],
                             add=True)

    prefetch(0, isA, idA, cfA)

    def pair(k, carry):
        gA = 2 * k
        gB = 2 * k + 1
        gA2 = jnp.minimum(gA + 2, NGRP - 1)
        phase(gA, gB, isA, idA, cfA, isB, idB, cfB)
        phase(gB, gA2, isB, idB, cfB, isA, idA, cfA)
        return carry

    lax.fori_loop(0, NGRP // 2, pair, 0)
    for b in range(G):
        pltpu.make_async_copy(rows_v.at[b], acc_sh.at[isB.at[b]],
                              sems[b]).wait()
    wait_slabs(NGRP - 1, isA, idA, cfA)
    plsc.subcore_barrier()
    wb = [pltpu.async_copy(acc_sh.at[pl.ds(nb + k * RZ, RZ)],
                           out_hbm.at[c, pl.ds(nb + k * RZ, RZ)], semz)
          for k in range(ROWS_T // RZ)]
    for dsc in wb:
        dsc.wait()


def _tc_dinv_body(deg_ref, out_ref):
    out_ref[...] = lax.rsqrt(deg_ref[0] + deg_ref[1] + 1.0)


def _tc_dinv(deg2):
    return pl.pallas_call(
        _tc_dinv_body,
        out_shape=jax.ShapeDtypeStruct((N_P // D, D), jnp.float32),
    )(deg2.reshape(2, N_P // D, D))


def _tc_q_body(d2_ref, dinv_ref, q_ref, q2_ref):
    dinv = dinv_ref[...]
    dd = jnp.maximum(d2_ref[0] + d2_ref[1] + dinv * dinv, 1e-12)
    q = dinv * lax.rsqrt(dd)
    q_ref[...] = q
    q2_ref[...] = q * q


def _tc_q(d2, dinv):
    return pl.pallas_call(
        _tc_q_body,
        out_shape=(
            jax.ShapeDtypeStruct((N_P // D, D), jnp.float32),
            jax.ShapeDtypeStruct((N_P // D, D), jnp.float32),
        ),
    )(d2.reshape(2, N_P // D, D), dinv)


_RB_TC = 1280


def _tc_linrelu_body(x_ref, w_ref, b_ref, out_ref):
    y = jnp.dot(x_ref[...], w_ref[...], preferred_element_type=jnp.float32)
    out_ref[...] = jnp.maximum(y + b_ref[...], 0.0)


def _tc_linrelu(x, w, b):
    return pl.pallas_call(
        _tc_linrelu_body,
        out_shape=jax.ShapeDtypeStruct((N_P, D), jnp.float32),
        grid=(N_P // _RB_TC,),
        in_specs=[
            pl.BlockSpec((_RB_TC, D), lambda i: (i, 0)),
            pl.BlockSpec((D, D), lambda i: (0, 0)),
            pl.BlockSpec((1, D), lambda i: (0, 0)),
        ],
        out_specs=pl.BlockSpec((_RB_TC, D), lambda i: (i, 0)),
    )(x, w, b.reshape(1, D))


def _combine(g0, g1, f, f0, q2):
    return ALPHA * (g0 + g1 + q2 * f) + BETA * f0


def _tc_combine_body(g0_ref, g1_ref, f_ref, f0_ref, q2_ref, out_ref):
    out_ref[...] = _combine(g0_ref[...], g1_ref[...], f_ref[...],
                            f0_ref[...], q2_ref[...])


def _tc_combine(g0, g1, f, f0, q2):
    return pl.pallas_call(
        _tc_combine_body,
        out_shape=jax.ShapeDtypeStruct((N_P, D), jnp.float32),
        grid=(N_P // _RB_TC,),
        in_specs=[
            pl.BlockSpec((_RB_TC, D), lambda i: (i, 0)),
            pl.BlockSpec((_RB_TC, D), lambda i: (i, 0)),
            pl.BlockSpec((_RB_TC, D), lambda i: (i, 0)),
            pl.BlockSpec((_RB_TC, D), lambda i: (i, 0)),
            pl.BlockSpec((_RB_TC, 1), lambda i: (i, 0)),
        ],
        out_specs=pl.BlockSpec((_RB_TC, D), lambda i: (i, 0)),
    )(g0, g1, f, f0, q2)


def _tc_comb_mm_relu_body(g0_ref, g1_ref, f_ref, f0_ref, q2_ref, w_ref, b_ref,
                          out_ref):
    z = _combine(g0_ref[...], g1_ref[...], f_ref[...], f0_ref[...], q2_ref[...])
    y = jnp.dot(z, w_ref[...], preferred_element_type=jnp.float32)
    out_ref[...] = jnp.maximum(y + b_ref[...], 0.0)


def _tc_comb_mm_relu(g0, g1, f, f0, q2, w, b):
    return pl.pallas_call(
        _tc_comb_mm_relu_body,
        out_shape=jax.ShapeDtypeStruct((N_P, D), jnp.float32),
        grid=(N_P // _RB_TC,),
        in_specs=[
            pl.BlockSpec((_RB_TC, D), lambda i: (i, 0)),
            pl.BlockSpec((_RB_TC, D), lambda i: (i, 0)),
            pl.BlockSpec((_RB_TC, D), lambda i: (i, 0)),
            pl.BlockSpec((_RB_TC, D), lambda i: (i, 0)),
            pl.BlockSpec((_RB_TC, 1), lambda i: (i, 0)),
            pl.BlockSpec((D, D), lambda i: (0, 0)),
            pl.BlockSpec((1, D), lambda i: (0, 0)),
        ],
        out_specs=pl.BlockSpec((_RB_TC, D), lambda i: (i, 0)),
    )(g0, g1, f, f0, q2, w, b.reshape(1, D))


_RB_LSM = 2000


def _tc_comb_mm_lsm_body(g0_ref, g1_ref, f_ref, f0_ref, q2_ref, w_ref, b_ref,
                         out_ref):
    z = _combine(g0_ref[...], g1_ref[...], f_ref[...], f0_ref[...], q2_ref[...])
    y = jnp.dot(z, w_ref[...], preferred_element_type=jnp.float32) + b_ref[...]
    m = jnp.max(y, axis=1, keepdims=True)
    lse = m + jnp.log(jnp.sum(jnp.exp(y - m), axis=1, keepdims=True))
    out_ref[...] = y - lse


def _tc_comb_mm_lsm(g0, g1, f, f0, q2, w, b):
    return pl.pallas_call(
        _tc_comb_mm_lsm_body,
        out_shape=jax.ShapeDtypeStruct((N, DO), jnp.float32),
        grid=(N // _RB_LSM,),
        in_specs=[
            pl.BlockSpec((_RB_LSM, D), lambda i: (i, 0)),
            pl.BlockSpec((_RB_LSM, D), lambda i: (i, 0)),
            pl.BlockSpec((_RB_LSM, D), lambda i: (i, 0)),
            pl.BlockSpec((_RB_LSM, D), lambda i: (i, 0)),
            pl.BlockSpec((_RB_LSM, 1), lambda i: (i, 0)),
            pl.BlockSpec((D, DO), lambda i: (0, 0)),
            pl.BlockSpec((1, DO), lambda i: (0, 0)),
        ],
        out_specs=pl.BlockSpec((_RB_LSM, DO), lambda i: (i, 0)),
    )(g0, g1, f, f0, q2, w, b.reshape(1, DO))


def kernel(x, edge_index, W_l1, b_l1, W_c1, b_c1, W_c2, b_c2):
    pad = (jnp.arange(E_PAD - E, dtype=jnp.int32) % (N_P - N)) + N
    src = jnp.concatenate([edge_index[0].astype(jnp.int32), pad])
    dst = jnp.concatenate([edge_index[1].astype(jnp.int32), pad])
    srcr = src.reshape(NW, NB, EB)
    dstr = dst.reshape(NW, NB, EB)
    srcg = src.reshape(NW, NGRP, G, EB)
    dstg = dst.reshape(NW, NGRP, G, EB)
    srcf = src.reshape(NW, E_TILE)
    dstf = dst.reshape(NW, E_TILE)
    xp = jnp.concatenate([x, jnp.zeros((N_P - N, D), jnp.float32)])

    deg2 = _sc_degree(dstr)
    dinv = _tc_dinv(deg2)
    d2 = _sc_dsum(srcr, dstr, dinv.reshape(N_P))
    q, q2f = _tc_q(d2, dinv)
    coef = _sc_coef(srcf, dstf, q.reshape(N_P))
    cfg = coef.reshape(NW, NGRP, G, EB)
    q2 = q2f.reshape(N_P, 1)

    f0 = _tc_linrelu(xp, W_l1, b_l1)

    g = _sc_spmv(srcg, dstg, cfg, f0)
    f1 = _tc_combine(g[0], g[1], f0, f0, q2)
    g = _sc_spmv(srcg, dstg, cfg, f1)
    h = _tc_comb_mm_relu(g[0], g[1], f1, f0, q2, W_c1, b_c1)

    g = _sc_spmv(srcg, dstg, cfg, h)
    f1 = _tc_combine(g[0], g[1], h, h, q2)
    g = _sc_spmv(srcg, dstg, cfg, f1)
    out = _tc_comb_mm_lsm(g[0], g[1], f1, h, q2, W_c2, b_c2)
    return out
